# baseline (speedup 1.0000x reference)
import functools

import jax
import jax.numpy as jnp
from jax import lax
from jax.experimental import pallas as pl
from jax.experimental.pallas import tpu as pltpu
from jax.experimental.pallas import tpu_sc as plsc

N = 10000
D = 128
E = 320000

NC, NS = 2, 16
NW = NC * NS
B = 128
CH = 8

ETOT = E + N
NB = -(-ETOT // (NW * B * CH)) * CH
NCH = NB // CH
TILE_E = NB * B
EPAD = TILE_E * NW
TOTB = EPAD // B
NB16 = 2 * NB

NPAD = 10240
NSLICE = NPAD // NS

_mesh = plsc.VectorSubcoreMesh(
    core_axis_name="c", subcore_axis_name="s", num_cores=NC, num_subcores=NS)


@functools.partial(
    pl.kernel,
    out_type=jax.ShapeDtypeStruct((NW, NB, B), jnp.float32),
    mesh=_mesh,
    compiler_params=pltpu.CompilerParams(needs_layout_passes=False),
    scratch_types=[
        pltpu.VMEM_SHARED((NS, NPAD), jnp.float32),
        pltpu.VMEM_SHARED((NPAD,), jnp.float32),
        pltpu.VMEM((NPAD,), jnp.float32),
        pltpu.VMEM((NB16, B), jnp.int32),
        pltpu.VMEM((NB16, B), jnp.float32),
        pltpu.VMEM((NB, B), jnp.int32),
        pltpu.VMEM((NB, B), jnp.float32),
        pltpu.VMEM((NSLICE,), jnp.float32),
        pltpu.VMEM((NSLICE,), jnp.float32),
        pltpu.VMEM((NPAD,), jnp.float32),
    ],
)
def _norm_kernel(rows_h, cols_h, ew_h, norm_h, degs_sh, dis_sh, degacc,
                 c16, w16, r32, nst, dv, tv, disf):
    c = lax.axis_index("c")
    s = lax.axis_index("s")
    z16 = jnp.zeros((16,), jnp.float32)

    w = s * NC + c
    pltpu.sync_copy(cols_h.at[s], c16)
    pltpu.sync_copy(ew_h.at[s], w16)
    pltpu.sync_copy(rows_h.at[w], r32)

    def zb(i, _):
        degacc[pl.ds(pl.multiple_of(i * 16, 16), 16)] = z16
        return 0
    lax.fori_loop(0, NPAD // 16, zb, 0)

    def dblk(i, _):
        for k in range(B // 16):
            ksl = pl.ds(k * 16, 16)
            plsc.addupdate_scatter(degacc, [c16[i, ksl]], w16[i, ksl])
        return 0
    lax.fori_loop(0, NB16, dblk, 0)

    pltpu.sync_copy(degacc, degs_sh.at[s])
    plsc.subcore_barrier()

    pltpu.sync_copy(degs_sh.at[0, pl.ds(s * NSLICE, NSLICE)], dv)
    for t in range(1, NS):
        pltpu.sync_copy(degs_sh.at[t, pl.ds(s * NSLICE, NSLICE)], tv)

        def radd(i, _):
            ksl = pl.ds(pl.multiple_of(i * 16, 16), 16)
            dv[ksl] = dv[ksl] + tv[ksl]
            return 0
        lax.fori_loop(0, NSLICE // 16, radd, 0)

    half = jnp.full((16,), 0.5, jnp.float32)
    th = jnp.full((16,), 1.5, jnp.float32)
    one = jnp.full((16,), 1.0, jnp.float32)

    def rs(i, _):
        ksl = pl.ds(pl.multiple_of(i * 16, 16), 16)
        d = jnp.maximum(dv[ksl], one)
        y = one / d
        hd = half * d
        for _ in range(24):
            y = y * (th - hd * y * y)
        dv[ksl] = y
        return 0
    lax.fori_loop(0, NSLICE // 16, rs, 0)
    pltpu.sync_copy(dv, dis_sh.at[pl.ds(s * NSLICE, NSLICE)])
    plsc.subcore_barrier()

    pltpu.sync_copy(dis_sh, disf)

    def nblk(i, _):
        for k in range(B // 16):
            ksl = pl.ds(k * 16, 16)
            a = plsc.load_gather(disf, [r32[i, ksl]])
            b = plsc.load_gather(disf, [c16[c * NB + i, ksl]])
            nst[i, ksl] = a * w16[c * NB + i, ksl] * b
        return 0
    lax.fori_loop(0, NB, nblk, 0)
    pltpu.sync_copy(nst, norm_h.at[w])


@functools.partial(
    pl.kernel,
    out_type=jax.ShapeDtypeStruct((NC, N, D), jnp.float32),
    mesh=_mesh,
    compiler_params=pltpu.CompilerParams(needs_layout_passes=False),
    scratch_types=[
        pltpu.VMEM_SHARED((N, D), jnp.float32),
        pltpu.VMEM((2, CH, B), jnp.int32),
        pltpu.VMEM((2, CH, B), jnp.int32),
        pltpu.VMEM((2, CH, B), jnp.float32),
        pltpu.VMEM((2, B, D), jnp.float32),
        pltpu.SemaphoreType.DMA((2,)),
        pltpu.SemaphoreType.DMA,
        pltpu.SemaphoreType.DMA,
    ],
)
def _mp_kernel(h_h, rows_h, cols_h, norm_h, out_h, acc_sh, ridx, cidx, nv,
               rbuf, semi, semg, sems):
    c = lax.axis_index("c")
    s = lax.axis_index("s")
    z16 = jnp.zeros((16,), jnp.float32)

    def zb(i, _):
        r = i // (D // 16)
        k = i % (D // 16)
        rbuf[0, r, pl.ds(pl.multiple_of(k * 16, 16), 16)] = z16
        return 0
    lax.fori_loop(0, B * D // 16, zb, 0)

    @pl.when(s < NS - 1)
    def _():
        for j in range(4):
            pltpu.sync_copy(rbuf.at[0], acc_sh.at[pl.ds(s * 624 + j * B, B)])
        pltpu.sync_copy(rbuf.at[0, pl.ds(0, 112)],
                        acc_sh.at[pl.ds(s * 624 + 4 * B, 112)])

    @pl.when(s == NS - 1)
    def _():
        for j in range(5):
            pltpu.sync_copy(rbuf.at[0], acc_sh.at[pl.ds(15 * 624 + j * B, B)])

    plsc.subcore_barrier()

    w = s * NC + c
    b0 = w * NB

    def chunk_load(cn):
        slot = cn % 2
        pltpu.async_copy(rows_h.at[pl.ds(b0 + cn * CH, CH)], ridx.at[slot],
                         semi.at[slot])
        pltpu.async_copy(cols_h.at[pl.ds(b0 + cn * CH, CH)], cidx.at[slot],
                         semi.at[slot])
        pltpu.async_copy(norm_h.at[pl.ds(b0 + cn * CH, CH)], nv.at[slot],
                         semi.at[slot])

    def chunk_wait(cn):
        slot = cn % 2
        pltpu.make_async_copy(rows_h.at[pl.ds(0, CH)], ridx.at[slot],
                              semi.at[slot]).wait()
        pltpu.make_async_copy(cols_h.at[pl.ds(0, CH)], cidx.at[slot],
                              semi.at[slot]).wait()
        pltpu.make_async_copy(norm_h.at[pl.ds(0, CH)], nv.at[slot],
                              semi.at[slot]).wait()

    pltpu.sync_copy(rows_h.at[pl.ds(b0, CH)], ridx.at[0])
    pltpu.sync_copy(cols_h.at[pl.ds(b0, CH)], cidx.at[0])
    pltpu.sync_copy(norm_h.at[pl.ds(b0, CH)], nv.at[0])
    pltpu.async_copy(h_h.at[ridx.at[0, 0]], rbuf.at[0], semg)

    def blk(i, _):
        sub = i % CH
        cn = i // CH
        j = i % 2

        pltpu.make_async_copy(h_h.at[ridx.at[0, 0]], rbuf.at[j], semg).wait()

        @plsc.parallel_loop(0, B, step=1, unroll=8)
        def scale(e):
            ns = plsc.load_gather(nv.at[cn % 2, sub],
                                  [jnp.full((16,), e, jnp.int32)])
            for kk in range(D // 16):
                ksl = pl.ds(kk * 16, 16)
                rbuf[j, e, ksl] = rbuf[j, e, ksl] * ns

        @pl.when(i >= 1)
        def _():
            pltpu.make_async_copy(rbuf.at[(i - 1) % 2],
                                  acc_sh.at[cidx.at[0, 0]], sems).wait()

        @pl.when(jnp.logical_and(sub == 0, cn + 1 < NCH))
        def _():
            chunk_load(cn + 1)

        @pl.when(i + 1 < NB)
        def _():
            @pl.when(sub == CH - 1)
            def _():
                chunk_wait(cn + 1)
            pltpu.async_copy(
                h_h.at[ridx.at[((i + 1) // CH) % 2, (i + 1) % CH]],
                rbuf.at[(i + 1) % 2], semg)

        pltpu.async_copy(rbuf.at[j], acc_sh.at[cidx.at[cn % 2, sub]],
                         sems, add=True)
        return 0
    lax.fori_loop(0, NB, blk, 0)

    pltpu.make_async_copy(rbuf.at[(NB - 1) % 2],
                          acc_sh.at[cidx.at[0, 0]], sems).wait()
    plsc.subcore_barrier()

    @pl.when(s < NS - 1)
    def _():
        pltpu.sync_copy(acc_sh.at[pl.ds(s * 624, 624)],
                        out_h.at[c, pl.ds(s * 624, 624)])

    @pl.when(s == NS - 1)
    def _():
        pltpu.sync_copy(acc_sh.at[pl.ds(15 * 624, 640)],
                        out_h.at[c, pl.ds(15 * 624, 640)])


def _mm_body(x_ref, w_ref, o_ref):
    o_ref[...] = lax.dot_general(
        x_ref[...], w_ref[...], (((1,), (1,)), ((), ())),
        preferred_element_type=jnp.float32)


def _tc_matmul(x, W):
    return pl.pallas_call(
        _mm_body,
        grid=(10,),
        in_specs=[pl.BlockSpec((N // 10, D), lambda i: (i, 0)),
                  pl.BlockSpec((D, D), lambda i: (0, 0))],
        out_specs=pl.BlockSpec((N // 10, D), lambda i: (i, 0)),
        out_shape=jax.ShapeDtypeStruct((N, D), jnp.float32),
    )(x, W)


def _mm2_body(p_ref, b_ref, w_ref, o_ref):
    t = jnp.maximum(p_ref[0] + p_ref[1] + b_ref[...], 0.0)
    o_ref[...] = lax.dot_general(
        t, w_ref[...], (((1,), (1,)), ((), ())),
        preferred_element_type=jnp.float32)


def _tc_combine_matmul(p, b, W):
    return pl.pallas_call(
        _mm2_body,
        grid=(10,),
        in_specs=[pl.BlockSpec((NC, N // 10, D), lambda i: (0, i, 0)),
                  pl.BlockSpec((1, D), lambda i: (0, 0)),
                  pl.BlockSpec((D, D), lambda i: (0, 0))],
        out_specs=pl.BlockSpec((N // 10, D), lambda i: (i, 0)),
        out_shape=jax.ShapeDtypeStruct((N, D), jnp.float32),
    )(p, b, W)


def _fin_body(p_ref, b_ref, o_ref):
    o_ref[...] = jnp.maximum(p_ref[0] + p_ref[1] + b_ref[...], 0.0)


def _tc_combine_relu(p, b):
    return pl.pallas_call(
        _fin_body,
        grid=(10,),
        in_specs=[pl.BlockSpec((NC, N // 10, D), lambda i: (0, i, 0)),
                  pl.BlockSpec((1, D), lambda i: (0, 0))],
        out_specs=pl.BlockSpec((N // 10, D), lambda i: (i, 0)),
        out_shape=jax.ShapeDtypeStruct((N, D), jnp.float32),
    )(p, b)


def kernel(x, edge_index, edge_weights, W1, b1, W2, b2):
    row = edge_index[0]
    col = edge_index[1]
    loop = jnp.arange(N, dtype=row.dtype)
    npad = EPAD - ETOT
    pad_idx = jnp.arange(npad, dtype=row.dtype) % N
    rows_flat = jnp.concatenate([row, loop, pad_idx])
    cols_flat = jnp.concatenate([col, loop, pad_idx])
    ew_flat = jnp.concatenate([edge_weights, jnp.ones((N,), jnp.float32),
                               jnp.zeros((npad,), jnp.float32)])
    rows3 = rows_flat.reshape(NW, NB, B)
    cols16 = cols_flat.reshape(NS, NB16, B)
    ew16 = ew_flat.reshape(NS, NB16, B)
    rows2 = rows_flat.reshape(TOTB, B)
    cols2 = cols_flat.reshape(TOTB, B)

    norm2 = _norm_kernel(rows3, cols16, ew16).reshape(TOTB, B)

    h1 = _tc_matmul(x, W1)
    p1 = _mp_kernel(h1, rows2, cols2, norm2)
    h2 = _tc_combine_matmul(p1, b1.reshape(1, D), W2)
    p2 = _mp_kernel(h2, rows2, cols2, norm2)
    return _tc_combine_relu(p2, b2.reshape(1, D))


# speedup vs baseline: 1.2626x; 1.2626x over previous
import functools

import jax
import jax.numpy as jnp
from jax import lax
from jax.experimental import pallas as pl
from jax.experimental.pallas import tpu as pltpu
from jax.experimental.pallas import tpu_sc as plsc

N = 10000
D = 128
E = 320000

NC, NS = 2, 16
NW = NC * NS
B = 128
CH = 8

ETOT = E + N
NB = -(-ETOT // (NW * B * CH)) * CH
NCH = NB // CH
TILE_E = NB * B
EPAD = TILE_E * NW
TOTB = EPAD // B
NB16 = 2 * NB

NPAD = 10240
NSLICE = NPAD // NS

_mesh = plsc.VectorSubcoreMesh(
    core_axis_name="c", subcore_axis_name="s", num_cores=NC, num_subcores=NS)


@functools.partial(
    pl.kernel,
    out_type=jax.ShapeDtypeStruct((NW, NB, B), jnp.float32),
    mesh=_mesh,
    compiler_params=pltpu.CompilerParams(needs_layout_passes=False),
    scratch_types=[
        pltpu.VMEM_SHARED((NS, NPAD), jnp.float32),
        pltpu.VMEM_SHARED((NPAD,), jnp.float32),
        pltpu.VMEM((NPAD,), jnp.float32),
        pltpu.VMEM((NB16, B), jnp.int32),
        pltpu.VMEM((NB16, B), jnp.float32),
        pltpu.VMEM((NB, B), jnp.int32),
        pltpu.VMEM((NB, B), jnp.float32),
        pltpu.VMEM((NSLICE,), jnp.float32),
        pltpu.VMEM((NSLICE,), jnp.float32),
        pltpu.VMEM((NPAD,), jnp.float32),
    ],
)
def _norm_kernel(rows_h, cols_h, ew_h, norm_h, degs_sh, dis_sh, degacc,
                 c16, w16, r32, nst, dv, tv, disf):
    c = lax.axis_index("c")
    s = lax.axis_index("s")
    z16 = jnp.zeros((16,), jnp.float32)

    w = s * NC + c
    pltpu.sync_copy(cols_h.at[s], c16)
    pltpu.sync_copy(ew_h.at[s], w16)
    pltpu.sync_copy(rows_h.at[w], r32)

    def zb(i, _):
        degacc[pl.ds(pl.multiple_of(i * 16, 16), 16)] = z16
        return 0
    lax.fori_loop(0, NPAD // 16, zb, 0)

    def dblk(i, _):
        for k in range(B // 16):
            ksl = pl.ds(k * 16, 16)
            plsc.addupdate_scatter(degacc, [c16[i, ksl]], w16[i, ksl])
        return 0
    lax.fori_loop(0, NB16, dblk, 0)

    pltpu.sync_copy(degacc, degs_sh.at[s])
    plsc.subcore_barrier()

    pltpu.sync_copy(degs_sh.at[0, pl.ds(s * NSLICE, NSLICE)], dv)
    for t in range(1, NS):
        pltpu.sync_copy(degs_sh.at[t, pl.ds(s * NSLICE, NSLICE)], tv)

        def radd(i, _):
            ksl = pl.ds(pl.multiple_of(i * 16, 16), 16)
            dv[ksl] = dv[ksl] + tv[ksl]
            return 0
        lax.fori_loop(0, NSLICE // 16, radd, 0)

    half = jnp.full((16,), 0.5, jnp.float32)
    th = jnp.full((16,), 1.5, jnp.float32)
    one = jnp.full((16,), 1.0, jnp.float32)

    def rs(i, _):
        ksl = pl.ds(pl.multiple_of(i * 16, 16), 16)
        d = jnp.maximum(dv[ksl], one)
        y = one / d
        hd = half * d
        for _ in range(24):
            y = y * (th - hd * y * y)
        dv[ksl] = y
        return 0
    lax.fori_loop(0, NSLICE // 16, rs, 0)
    pltpu.sync_copy(dv, dis_sh.at[pl.ds(s * NSLICE, NSLICE)])
    plsc.subcore_barrier()

    pltpu.sync_copy(dis_sh, disf)

    def nblk(i, _):
        for k in range(B // 16):
            ksl = pl.ds(k * 16, 16)
            a = plsc.load_gather(disf, [r32[i, ksl]])
            b = plsc.load_gather(disf, [c16[c * NB + i, ksl]])
            nst[i, ksl] = a * w16[c * NB + i, ksl] * b
        return 0
    lax.fori_loop(0, NB, nblk, 0)
    pltpu.sync_copy(nst, norm_h.at[w])


@functools.partial(
    pl.kernel,
    out_type=jax.ShapeDtypeStruct((NC, N, D), jnp.float32),
    mesh=_mesh,
    compiler_params=pltpu.CompilerParams(needs_layout_passes=False,
                                         use_tc_tiling_on_sc=False),
    scratch_types=[
        pltpu.VMEM_SHARED((N, D), jnp.float32),
        pltpu.VMEM((2, CH, B), jnp.int32),
        pltpu.VMEM((2, CH, B), jnp.int32),
        pltpu.VMEM((2, CH, B), jnp.float32),
        pltpu.VMEM((2, B, D // 2), jnp.int32),
        pltpu.VMEM((B, D), jnp.float32),
        pltpu.SemaphoreType.DMA((2,)),
        pltpu.SemaphoreType.DMA,
        pltpu.SemaphoreType.DMA,
    ],
)
def _mp_kernel(h_h, rows_h, cols_h, norm_h, out_h, acc_sh, ridx, cidx, nv,
               rbuf, sbuf, semi, semg, sems):
    c = lax.axis_index("c")
    s = lax.axis_index("s")
    z16 = jnp.zeros((16,), jnp.float32)

    def zb(i, _):
        r = i // (D // 16)
        k = i % (D // 16)
        sbuf[r, pl.ds(pl.multiple_of(k * 16, 16), 16)] = z16
        return 0
    lax.fori_loop(0, B * D // 16, zb, 0)

    @pl.when(s < NS - 1)
    def _():
        for j in range(4):
            pltpu.sync_copy(sbuf, acc_sh.at[pl.ds(s * 624 + j * B, B)])
        pltpu.sync_copy(sbuf.at[pl.ds(0, 112)],
                        acc_sh.at[pl.ds(s * 624 + 4 * B, 112)])

    @pl.when(s == NS - 1)
    def _():
        for j in range(5):
            pltpu.sync_copy(sbuf, acc_sh.at[pl.ds(15 * 624 + j * B, B)])

    plsc.subcore_barrier()

    w = s * NC + c
    b0 = w * NB

    def chunk_load(cn):
        slot = cn % 2
        pltpu.async_copy(rows_h.at[pl.ds(b0 + cn * CH, CH)], ridx.at[slot],
                         semi.at[slot])
        pltpu.async_copy(cols_h.at[pl.ds(b0 + cn * CH, CH)], cidx.at[slot],
                         semi.at[slot])
        pltpu.async_copy(norm_h.at[pl.ds(b0 + cn * CH, CH)], nv.at[slot],
                         semi.at[slot])

    def chunk_wait(cn):
        slot = cn % 2
        pltpu.make_async_copy(rows_h.at[pl.ds(0, CH)], ridx.at[slot],
                              semi.at[slot]).wait()
        pltpu.make_async_copy(cols_h.at[pl.ds(0, CH)], cidx.at[slot],
                              semi.at[slot]).wait()
        pltpu.make_async_copy(norm_h.at[pl.ds(0, CH)], nv.at[slot],
                              semi.at[slot]).wait()

    pltpu.sync_copy(rows_h.at[pl.ds(b0, CH)], ridx.at[0])
    pltpu.sync_copy(cols_h.at[pl.ds(b0, CH)], cidx.at[0])
    pltpu.sync_copy(norm_h.at[pl.ds(b0, CH)], nv.at[0])
    pltpu.async_copy(h_h.at[ridx.at[0, 0]], rbuf.at[0], semg)

    def blk(i, _):
        sub = i % CH
        cn = i // CH
        j = i % 2

        @pl.when(i + 1 < NB)
        def _():
            @pl.when(sub == CH - 1)
            def _():
                chunk_wait(cn + 1)
            pltpu.async_copy(
                h_h.at[ridx.at[((i + 1) // CH) % 2, (i + 1) % CH]],
                rbuf.at[(i + 1) % 2], semg)

        pltpu.make_async_copy(h_h.at[ridx.at[0, 0]], rbuf.at[j], semg).wait()

        @pl.when(i >= 1)
        def _():
            pltpu.make_async_copy(sbuf, acc_sh.at[cidx.at[0, 0]],
                                  sems).wait()

        @pl.when(jnp.logical_and(sub == 0, cn + 1 < NCH))
        def _():
            chunk_load(cn + 1)

        @plsc.parallel_loop(0, B, step=1, unroll=8)
        def scale(e):
            ns = plsc.load_gather(nv.at[cn % 2, sub],
                                  [jnp.full((16,), e, jnp.int32)])
            for g in range(D // 32):
                v = plsc.bitcast(rbuf[j, e, pl.ds(g * 16, 16)], jnp.bfloat16)
                a, b = plsc.unpack(v, format=plsc.PackFormat.INTERLEAVED)
                sbuf[e, pl.ds(g * 32, 16)] = a * ns
                sbuf[e, pl.ds(g * 32 + 16, 16)] = b * ns

        pltpu.async_copy(sbuf, acc_sh.at[cidx.at[cn % 2, sub]],
                         sems, add=True)
        return 0
    lax.fori_loop(0, NB, blk, 0)

    pltpu.make_async_copy(sbuf, acc_sh.at[cidx.at[0, 0]], sems).wait()
    plsc.subcore_barrier()

    @pl.when(s < NS - 1)
    def _():
        pltpu.sync_copy(acc_sh.at[pl.ds(s * 624, 624)],
                        out_h.at[c, pl.ds(s * 624, 624)])

    @pl.when(s == NS - 1)
    def _():
        pltpu.sync_copy(acc_sh.at[pl.ds(15 * 624, 640)],
                        out_h.at[c, pl.ds(15 * 624, 640)])


def _mm_body(x_ref, w_ref, o_ref):
    o_ref[...] = lax.dot_general(
        x_ref[...], w_ref[...], (((1,), (1,)), ((), ())),
        preferred_element_type=jnp.float32).astype(jnp.bfloat16)


def _tc_matmul(x, W):
    return pl.pallas_call(
        _mm_body,
        grid=(10,),
        in_specs=[pl.BlockSpec((N // 10, D), lambda i: (i, 0)),
                  pl.BlockSpec((D, D), lambda i: (0, 0))],
        out_specs=pl.BlockSpec((N // 10, D), lambda i: (i, 0)),
        out_shape=jax.ShapeDtypeStruct((N, D), jnp.bfloat16),
    )(x, W)


def _mm2_body(p_ref, b_ref, w_ref, o_ref):
    t = jnp.maximum(p_ref[0] + p_ref[1] + b_ref[...], 0.0)
    o_ref[...] = lax.dot_general(
        t, w_ref[...], (((1,), (1,)), ((), ())),
        preferred_element_type=jnp.float32).astype(jnp.bfloat16)


def _tc_combine_matmul(p, b, W):
    return pl.pallas_call(
        _mm2_body,
        grid=(10,),
        in_specs=[pl.BlockSpec((NC, N // 10, D), lambda i: (0, i, 0)),
                  pl.BlockSpec((1, D), lambda i: (0, 0)),
                  pl.BlockSpec((D, D), lambda i: (0, 0))],
        out_specs=pl.BlockSpec((N // 10, D), lambda i: (i, 0)),
        out_shape=jax.ShapeDtypeStruct((N, D), jnp.bfloat16),
    )(p, b, W)


def _fin_body(p_ref, b_ref, o_ref):
    o_ref[...] = jnp.maximum(p_ref[0] + p_ref[1] + b_ref[...], 0.0)


def _tc_combine_relu(p, b):
    return pl.pallas_call(
        _fin_body,
        grid=(10,),
        in_specs=[pl.BlockSpec((NC, N // 10, D), lambda i: (0, i, 0)),
                  pl.BlockSpec((1, D), lambda i: (0, 0))],
        out_specs=pl.BlockSpec((N // 10, D), lambda i: (i, 0)),
        out_shape=jax.ShapeDtypeStruct((N, D), jnp.float32),
    )(p, b)


def kernel(x, edge_index, edge_weights, W1, b1, W2, b2):
    row = edge_index[0]
    col = edge_index[1]
    loop = jnp.arange(N, dtype=row.dtype)
    npad = EPAD - ETOT
    pad_idx = jnp.arange(npad, dtype=row.dtype) % N
    rows_flat = jnp.concatenate([row, loop, pad_idx])
    cols_flat = jnp.concatenate([col, loop, pad_idx])
    ew_flat = jnp.concatenate([edge_weights, jnp.ones((N,), jnp.float32),
                               jnp.zeros((npad,), jnp.float32)])
    rows3 = rows_flat.reshape(NW, NB, B)
    cols16 = cols_flat.reshape(NS, NB16, B)
    ew16 = ew_flat.reshape(NS, NB16, B)
    rows2 = rows_flat.reshape(TOTB, B)
    cols2 = cols_flat.reshape(TOTB, B)

    norm2 = _norm_kernel(rows3, cols16, ew16).reshape(TOTB, B)

    perm = []
    for g in range(D // 32):
        for m in range(16):
            perm.extend([32 * g + m, 32 * g + 16 + m])
    perm = jnp.array(perm, dtype=jnp.int32)

    def as_words(h):
        return lax.bitcast_convert_type(h.reshape(N, D // 2, 2), jnp.int32)

    h1 = as_words(_tc_matmul(x, W1[perm]))
    p1 = _mp_kernel(h1, rows2, cols2, norm2)
    h2 = as_words(_tc_combine_matmul(p1, b1.reshape(1, D), W2[perm]))
    p2 = _mp_kernel(h2, rows2, cols2, norm2)
    return _tc_combine_relu(p2, b2.reshape(1, D))
